# baseline (speedup 1.0000x reference)
import jax
import jax.numpy as jnp
from jax import lax
from jax.experimental import pallas as pl
from jax.experimental.pallas import tpu as pltpu
from jax.experimental.pallas import tpu_sc as plsc

_N = 2048
_E = 32768
_DI = 128
_DO = 128
_EPS = 1e-06

_NC = 2
_NS = 16
_L = 16
_NW = _NC * _NS
_EW = _E // _NW
_CH = 16
_NCH = _EW // _CH
_NRS = _E // _NS // _CH
_NV = _DO // _L
_DR = _N // _L // _L


def _tc_pre(x_ref, fw_ref, ww_ref, ys_ref, yt_ref, av_ref):
    x = x_ref[...]
    fw = fw_ref[...]
    dn = (((1,), (1,)), ((), ()))
    ys_ref[...] = lax.dot_general(x, fw[:, :_DI], dn,
                                  preferred_element_type=jnp.float32)
    yt_ref[...] = lax.dot_general(x, fw[:, _DI:], dn,
                                  preferred_element_type=jnp.float32)
    av_ref[...] = lax.dot_general(ww_ref[...], x, dn,
                                  preferred_element_type=jnp.float32)


def _tc_post(num_ref, den_ref, o_ref):
    num = num_ref[0] + num_ref[1]
    den = den_ref[0] + den_ref[1] + _EPS
    o_ref[...] = num / den


def _sc_edge(ys_hbm, yt_hbm, av_hbm, src_hbm, tgt_hbm, tgt3_hbm, fb_hbm,
             num_hbm, den_hbm,
             asv_v, atv_v, srcA_v, tgtA_v, tgt3_v, alin_v, fb_v,
             ybuf0, ybuf1, ybuf2, ybuf3, ytbuf0, ytbuf1, ytbuf2, ytbuf3,
             obuf0, obuf1, obuf2, obuf3, den_l, idx_v,
             maxv_v, maxall_v,
             acc_sh, den_sh, max_sh,
             gsem0, gsem1, gsem2, gsem3, ssem0, ssem1, ssem2, ssem3):
    c = lax.axis_index("c")
    s = lax.axis_index("s")

    pltpu.sync_copy(av_hbm.at[0], asv_v)
    pltpu.sync_copy(av_hbm.at[1], atv_v)
    pltpu.sync_copy(fb_hbm, fb_v)
    pltpu.sync_copy(src_hbm.at[pl.ds(s * _NRS, _NRS)], srcA_v)
    pltpu.sync_copy(tgt_hbm.at[pl.ds(s * _NRS, _NRS)], tgtA_v)
    pltpu.sync_copy(tgt3_hbm.at[pl.ds(s * _NRS, _NRS)], tgt3_v)

    z = jnp.zeros((_L,), jnp.float32)
    idx_v[...] = lax.iota(jnp.int32, _L)

    def _zero(i, carry):
        for v in range(_NV):
            obuf0[i, pl.ds(v * _L, _L)] = z
        return carry

    lax.fori_loop(0, _CH, _zero, 0)

    def _zden(i, carry):
        for v in range(_DR):
            den_l[i, v, :] = z
        return carry

    lax.fori_loop(0, _L, _zden, 0)

    rows = _N // _NS
    for q in range(rows // _CH):
        pltpu.sync_copy(obuf0, acc_sh.at[pl.ds(s * rows + q * _CH, _CH)])

    @pl.when(s == 0)
    def _init_den():
        pltpu.sync_copy(obuf0.at[pl.ds(0, _L)], den_sh)

    def _stage_a(r, mx):
        for j in range(_CH // _L):
            si = srcA_v[r, pl.ds(j * _L, _L)]
            ti = tgtA_v[r, pl.ds(j * _L, _L)]
            a = (plsc.load_gather(asv_v, [si]) +
                 plsc.load_gather(atv_v, [ti]))
            alin_v[r, j, :] = a
            mx = jnp.maximum(mx, a)
        return mx

    maxv = lax.fori_loop(0, _NRS, _stage_a,
                         jnp.full((_L,), -jnp.inf, jnp.float32))
    maxv_v[...] = maxv
    pltpu.sync_copy(maxv_v, max_sh.at[s])

    plsc.subcore_barrier()

    pltpu.sync_copy(max_sh, maxall_v)
    mx = maxall_v[0, :]
    for r in range(1, _NS):
        mx = jnp.maximum(mx, maxall_v[r, :])
    gmax = jnp.max(mx)

    fbv = [fb_v[pl.ds(v * _L, _L)] for v in range(_NV)]
    iota16 = lax.iota(jnp.int32, _L)

    ybufs = (ybuf0, ybuf1, ybuf2, ybuf3)
    ytbufs = (ytbuf0, ytbuf1, ytbuf2, ytbuf3)
    obufs = (obuf0, obuf1, obuf2, obuf3)
    gsems = (gsem0, gsem1, gsem2, gsem3)
    ssems = (ssem0, ssem1, ssem2, ssem3)
    base = c * _NCH

    def _issue_gather(b, kr):
        pltpu.async_copy(ys_hbm.at[srcA_v.at[kr]], ybufs[b], gsems[b])
        pltpu.async_copy(yt_hbm.at[tgtA_v.at[kr]], ytbufs[b], gsems[b])

    def _drain_gather(b, kr):
        pltpu.make_async_copy(ys_hbm.at[srcA_v.at[kr]], ybufs[b],
                              gsems[b]).wait()
        pltpu.make_async_copy(yt_hbm.at[tgtA_v.at[kr]], ytbufs[b],
                              gsems[b]).wait()

    def _issue_scatter(b, kr):
        pltpu.async_copy(obufs[b], acc_sh.at[tgtA_v.at[kr]], ssems[b],
                         add=True)

    def _drain_scatter(b, kr):
        pltpu.make_async_copy(obufs[b], acc_sh.at[tgtA_v.at[kr]],
                              ssems[b]).wait()

    for b0 in range(4):
        _issue_gather(b0, base + b0)

    def _quad(pi, carry):
        for b in range(4):
            k = 4 * pi + b
            kr = base + k
            _drain_gather(b, kr)

            @pl.when(k >= 4)
            def _w1():
                _drain_scatter(b, kr - 4)

            yb = ybufs[b]
            tb = ytbufs[b]
            ob = obufs[b]

            def _load(e):
                return ([yb[e, pl.ds(v * _L, _L)] for v in range(_NV)] +
                        [tb[e, pl.ds(v * _L, _L)] for v in range(_NV)])

            def _group(g, carry2):
                ae16 = jnp.exp(alin_v[kr, g, :] - gmax)
                tv16 = tgt3_v[kr, g, :]
                cur = _load(g * _L)
                for kk in range(_L):
                    e = g * _L + kk
                    nxt = _load(e + 1) if kk < _L - 1 else cur
                    ae = ae16[kk]
                    for v in range(_NV):
                        yv = cur[v] + cur[_NV + v] + fbv[v]
                        ob[e, pl.ds(v * _L, _L)] = jnp.maximum(yv, 0.0) * ae
                    t = tv16[kk]
                    r = lax.shift_right_logical(t, 7)
                    sub = lax.bitwise_and(lax.shift_right_logical(t, 4), 7)
                    lane = lax.bitwise_and(t, 15)
                    oh = jnp.where(iota16 == lane, ae, 0.0)
                    den_l[r, sub, :] = den_l[r, sub, :] + oh
                    cur = nxt
                return carry2

            lax.fori_loop(0, _CH // _L, _group, 0)

            @pl.when(k + 4 < _NCH)
            def _w2():
                _issue_gather(b, kr + 4)

            _issue_scatter(b, kr)
        return carry

    lax.fori_loop(0, _NCH // 4, _quad, 0)
    for b0 in range(4):
        _drain_scatter(b0, base + _NCH - 4 + b0)

    def _pack(r, carry):
        for v in range(_DR):
            obuf1[r, pl.ds(v * _L, _L)] = den_l[r, v, :]
        return carry

    lax.fori_loop(0, _L, _pack, 0)
    pltpu.sync_copy(obuf1.at[pl.ds(0, _L)], den_sh.at[idx_v], add=True)

    plsc.subcore_barrier()

    pltpu.sync_copy(acc_sh.at[pl.ds(s * rows, rows)],
                    num_hbm.at[c, pl.ds(s * rows, rows)])

    @pl.when(s == 0)
    def _out_den():
        pltpu.sync_copy(den_sh, den_hbm.at[c])


_sc_call = pl.kernel(
    _sc_edge,
    out_type=(jax.ShapeDtypeStruct((_NC, _N, _DO), jnp.float32),
              jax.ShapeDtypeStruct((_NC, _L, _DO), jnp.float32)),
    mesh=plsc.VectorSubcoreMesh(core_axis_name="c", subcore_axis_name="s"),
    compiler_params=pltpu.CompilerParams(needs_layout_passes=False),
    scratch_types=[
        pltpu.VMEM((_N,), jnp.float32),
        pltpu.VMEM((_N,), jnp.float32),
        pltpu.VMEM((_NRS, _CH), jnp.int32),
        pltpu.VMEM((_NRS, _CH), jnp.int32),
        pltpu.VMEM((_NRS, _CH // _L, _L), jnp.int32),
        pltpu.VMEM((_NRS, _CH // _L, _L), jnp.float32),
        pltpu.VMEM((_DO,), jnp.float32),
        pltpu.VMEM((_CH, _DO), jnp.float32),
        pltpu.VMEM((_CH, _DO), jnp.float32),
        pltpu.VMEM((_CH, _DO), jnp.float32),
        pltpu.VMEM((_CH, _DO), jnp.float32),
        pltpu.VMEM((_CH, _DO), jnp.float32),
        pltpu.VMEM((_CH, _DO), jnp.float32),
        pltpu.VMEM((_CH, _DO), jnp.float32),
        pltpu.VMEM((_CH, _DO), jnp.float32),
        pltpu.VMEM((_CH, _DO), jnp.float32),
        pltpu.VMEM((_CH, _DO), jnp.float32),
        pltpu.VMEM((_CH, _DO), jnp.float32),
        pltpu.VMEM((_CH, _DO), jnp.float32),
        pltpu.VMEM((_L, _DR, _L), jnp.float32),
        pltpu.VMEM((_L,), jnp.int32),
        pltpu.VMEM((_L,), jnp.float32),
        pltpu.VMEM((_NS, _L), jnp.float32),
        pltpu.VMEM_SHARED((_N, _DO), jnp.float32),
        pltpu.VMEM_SHARED((_L, _DO), jnp.float32),
        pltpu.VMEM_SHARED((_NS, _L), jnp.float32),
        pltpu.SemaphoreType.DMA,
        pltpu.SemaphoreType.DMA,
        pltpu.SemaphoreType.DMA,
        pltpu.SemaphoreType.DMA,
        pltpu.SemaphoreType.DMA,
        pltpu.SemaphoreType.DMA,
        pltpu.SemaphoreType.DMA,
        pltpu.SemaphoreType.DMA,
    ],
)


def kernel(x, adj, src, tgt, Msrc, Mtgt, f_w, f_b, w_w, w_b):
    src32 = src.astype(jnp.int32)
    tgt32 = tgt.astype(jnp.int32)
    src2 = src32.reshape(_E // _CH, _CH)
    tgt2 = tgt32.reshape(_E // _CH, _CH)
    tgt3 = tgt32.reshape(_E // _CH, _CH // _L, _L)
    ww2 = w_w.reshape(2, _DI)
    ys, yt, av = pl.pallas_call(
        _tc_pre,
        out_shape=[
            jax.ShapeDtypeStruct((_N, _DO), jnp.float32),
            jax.ShapeDtypeStruct((_N, _DO), jnp.float32),
            jax.ShapeDtypeStruct((2, _N), jnp.float32),
        ],
    )(x, f_w, ww2)
    num_parts, den_parts = _sc_call(ys, yt, av, src2, tgt2, tgt3, f_b)
    den_col = den_parts.reshape(_NC, _N, 1)
    o = pl.pallas_call(
        _tc_post,
        out_shape=jax.ShapeDtypeStruct((_N, _DO), jnp.float32),
    )(num_parts, den_col)
    return o


# speedup vs baseline: 1.0893x; 1.0893x over previous
import jax
import jax.numpy as jnp
from jax import lax
from jax.experimental import pallas as pl
from jax.experimental.pallas import tpu as pltpu
from jax.experimental.pallas import tpu_sc as plsc

_N = 2048
_E = 32768
_DI = 128
_DO = 128
_EPS = 1e-06

_NC = 2
_NS = 16
_L = 16
_NW = _NC * _NS
_EW = _E // _NW
_CH = 64
_NCH = _EW // _CH
_NRS = _E // _NS // _CH
_NV = _DO // _L
_DR = _N // _L // _L


def _tc_pre(x_ref, fw_ref, ww_ref, ys_ref, yt_ref, av_ref):
    x = x_ref[...]
    fw = fw_ref[...]
    dn = (((1,), (1,)), ((), ()))
    ys_ref[...] = lax.dot_general(x, fw[:, :_DI], dn,
                                  preferred_element_type=jnp.float32)
    yt_ref[...] = lax.dot_general(x, fw[:, _DI:], dn,
                                  preferred_element_type=jnp.float32)
    av_ref[...] = lax.dot_general(ww_ref[...], x, dn,
                                  preferred_element_type=jnp.float32)


def _tc_post(num_ref, den_ref, o_ref):
    num = num_ref[0] + num_ref[1]
    den = den_ref[0] + den_ref[1] + _EPS
    o_ref[...] = num / den


def _sc_edge(ys_hbm, yt_hbm, av_hbm, src_hbm, tgt_hbm, tgt3_hbm, fb_hbm,
             num_hbm, den_hbm,
             asv_v, atv_v, srcA_v, tgtA_v, tgt3_v, alin_v, fb_v,
             ybuf0, ybuf1, ytbuf0, ytbuf1, obuf0, obuf1, den_l, idx_v,
             maxv_v, maxall_v,
             acc_sh, den_sh, max_sh, gsem0, gsem1, ssem0, ssem1):
    c = lax.axis_index("c")
    s = lax.axis_index("s")

    pltpu.sync_copy(av_hbm.at[0], asv_v)
    pltpu.sync_copy(av_hbm.at[1], atv_v)
    pltpu.sync_copy(fb_hbm, fb_v)
    pltpu.sync_copy(src_hbm.at[pl.ds(s * _NRS, _NRS)], srcA_v)
    pltpu.sync_copy(tgt_hbm.at[pl.ds(s * _NRS, _NRS)], tgtA_v)
    pltpu.sync_copy(tgt3_hbm.at[pl.ds(s * _NRS, _NRS)], tgt3_v)

    z = jnp.zeros((_L,), jnp.float32)
    idx_v[...] = lax.iota(jnp.int32, _L)

    def _zero(i, carry):
        for v in range(_NV):
            obuf0[i, pl.ds(v * _L, _L)] = z
        return carry

    lax.fori_loop(0, _CH, _zero, 0)

    def _zden(i, carry):
        for v in range(_DR):
            den_l[i, v, :] = z
        return carry

    lax.fori_loop(0, _L, _zden, 0)

    rows = _N // _NS
    for q in range(rows // _CH):
        pltpu.sync_copy(obuf0, acc_sh.at[pl.ds(s * rows + q * _CH, _CH)])

    @pl.when(s == 0)
    def _init_den():
        pltpu.sync_copy(obuf0.at[pl.ds(0, _L)], den_sh)

    def _stage_a(r, mx):
        for j in range(_CH // _L):
            si = srcA_v[r, pl.ds(j * _L, _L)]
            ti = tgtA_v[r, pl.ds(j * _L, _L)]
            a = (plsc.load_gather(asv_v, [si]) +
                 plsc.load_gather(atv_v, [ti]))
            alin_v[r, j, :] = a
            mx = jnp.maximum(mx, a)
        return mx

    maxv = lax.fori_loop(0, _NRS, _stage_a,
                         jnp.full((_L,), -jnp.inf, jnp.float32))
    maxv_v[...] = maxv
    pltpu.sync_copy(maxv_v, max_sh.at[s])

    plsc.subcore_barrier()

    pltpu.sync_copy(max_sh, maxall_v)
    mx = maxall_v[0, :]
    for r in range(1, _NS):
        mx = jnp.maximum(mx, maxall_v[r, :])
    gmax = jnp.max(mx)

    fbv = [fb_v[pl.ds(v * _L, _L)] for v in range(_NV)]
    iota16 = lax.iota(jnp.int32, _L)

    ybufs = (ybuf0, ybuf1)
    ytbufs = (ytbuf0, ytbuf1)
    obufs = (obuf0, obuf1)
    gsems = (gsem0, gsem1)
    ssems = (ssem0, ssem1)
    base = c * _NCH

    def _issue_gather(b, kr):
        pltpu.async_copy(ys_hbm.at[srcA_v.at[kr]], ybufs[b], gsems[b])
        pltpu.async_copy(yt_hbm.at[tgtA_v.at[kr]], ytbufs[b], gsems[b])

    def _drain_gather(b, kr):
        pltpu.make_async_copy(ys_hbm.at[srcA_v.at[kr]], ybufs[b],
                              gsems[b]).wait()
        pltpu.make_async_copy(yt_hbm.at[tgtA_v.at[kr]], ytbufs[b],
                              gsems[b]).wait()

    def _issue_scatter(b, kr):
        pltpu.async_copy(obufs[b], acc_sh.at[tgtA_v.at[kr]], ssems[b],
                         add=True)

    def _drain_scatter(b, kr):
        pltpu.make_async_copy(obufs[b], acc_sh.at[tgtA_v.at[kr]],
                              ssems[b]).wait()

    _issue_gather(0, base)
    _issue_gather(1, base + 1)

    def _pair(pi, carry):
        for b in range(2):
            k = 2 * pi + b
            kr = base + k
            _drain_gather(b, kr)

            @pl.when(k >= 2)
            def _w1():
                _drain_scatter(b, kr - 2)

            yb = ybufs[b]
            tb = ytbufs[b]
            ob = obufs[b]

            def _load(e):
                return ([yb[e, pl.ds(v * _L, _L)] for v in range(_NV)] +
                        [tb[e, pl.ds(v * _L, _L)] for v in range(_NV)])

            def _group(g, carry2):
                ae16 = jnp.exp(alin_v[kr, g, :] - gmax)
                tv16 = tgt3_v[kr, g, :]
                cur = _load(g * _L)
                for kk in range(_L):
                    e = g * _L + kk
                    nxt = _load(e + 1) if kk < _L - 1 else cur
                    ae = ae16[kk]
                    for v in range(_NV):
                        yv = cur[v] + cur[_NV + v] + fbv[v]
                        ob[e, pl.ds(v * _L, _L)] = jnp.maximum(yv, 0.0) * ae
                    t = tv16[kk]
                    r = lax.shift_right_logical(t, 7)
                    sub = lax.bitwise_and(lax.shift_right_logical(t, 4), 7)
                    lane = lax.bitwise_and(t, 15)
                    oh = jnp.where(iota16 == lane, ae, 0.0)
                    den_l[r, sub, :] = den_l[r, sub, :] + oh
                    cur = nxt
                return carry2

            lax.fori_loop(0, _CH // _L, _group, 0)

            @pl.when(k + 2 < _NCH)
            def _w2():
                _issue_gather(b, kr + 2)

            _issue_scatter(b, kr)
        return carry

    lax.fori_loop(0, _NCH // 2, _pair, 0)
    _drain_scatter(0, base + _NCH - 2)
    _drain_scatter(1, base + _NCH - 1)

    def _pack(r, carry):
        for v in range(_DR):
            obuf1[r, pl.ds(v * _L, _L)] = den_l[r, v, :]
        return carry

    lax.fori_loop(0, _L, _pack, 0)
    pltpu.sync_copy(obuf1.at[pl.ds(0, _L)], den_sh.at[idx_v], add=True)

    plsc.subcore_barrier()

    pltpu.sync_copy(acc_sh.at[pl.ds(s * rows, rows)],
                    num_hbm.at[c, pl.ds(s * rows, rows)])

    @pl.when(s == 0)
    def _out_den():
        pltpu.sync_copy(den_sh, den_hbm.at[c])


_sc_call = pl.kernel(
    _sc_edge,
    out_type=(jax.ShapeDtypeStruct((_NC, _N, _DO), jnp.float32),
              jax.ShapeDtypeStruct((_NC, _L, _DO), jnp.float32)),
    mesh=plsc.VectorSubcoreMesh(core_axis_name="c", subcore_axis_name="s"),
    compiler_params=pltpu.CompilerParams(needs_layout_passes=False),
    scratch_types=[
        pltpu.VMEM((_N,), jnp.float32),
        pltpu.VMEM((_N,), jnp.float32),
        pltpu.VMEM((_NRS, _CH), jnp.int32),
        pltpu.VMEM((_NRS, _CH), jnp.int32),
        pltpu.VMEM((_NRS, _CH // _L, _L), jnp.int32),
        pltpu.VMEM((_NRS, _CH // _L, _L), jnp.float32),
        pltpu.VMEM((_DO,), jnp.float32),
        pltpu.VMEM((_CH, _DO), jnp.float32),
        pltpu.VMEM((_CH, _DO), jnp.float32),
        pltpu.VMEM((_CH, _DO), jnp.float32),
        pltpu.VMEM((_CH, _DO), jnp.float32),
        pltpu.VMEM((_CH, _DO), jnp.float32),
        pltpu.VMEM((_CH, _DO), jnp.float32),
        pltpu.VMEM((_L, _DR, _L), jnp.float32),
        pltpu.VMEM((_L,), jnp.int32),
        pltpu.VMEM((_L,), jnp.float32),
        pltpu.VMEM((_NS, _L), jnp.float32),
        pltpu.VMEM_SHARED((_N, _DO), jnp.float32),
        pltpu.VMEM_SHARED((_L, _DO), jnp.float32),
        pltpu.VMEM_SHARED((_NS, _L), jnp.float32),
        pltpu.SemaphoreType.DMA,
        pltpu.SemaphoreType.DMA,
        pltpu.SemaphoreType.DMA,
        pltpu.SemaphoreType.DMA,
    ],
)


def kernel(x, adj, src, tgt, Msrc, Mtgt, f_w, f_b, w_w, w_b):
    src32 = src.astype(jnp.int32)
    tgt32 = tgt.astype(jnp.int32)
    src2 = src32.reshape(_E // _CH, _CH)
    tgt2 = tgt32.reshape(_E // _CH, _CH)
    tgt3 = tgt32.reshape(_E // _CH, _CH // _L, _L)
    ww2 = w_w.reshape(2, _DI)
    ys, yt, av = pl.pallas_call(
        _tc_pre,
        out_shape=[
            jax.ShapeDtypeStruct((_N, _DO), jnp.float32),
            jax.ShapeDtypeStruct((_N, _DO), jnp.float32),
            jax.ShapeDtypeStruct((2, _N), jnp.float32),
        ],
    )(x, f_w, ww2)
    num_parts, den_parts = _sc_call(ys, yt, av, src2, tgt2, tgt3, f_b)
    den_col = den_parts.reshape(_NC, _N, 1)
    o = pl.pallas_call(
        _tc_post,
        out_shape=jax.ShapeDtypeStruct((_N, _DO), jnp.float32),
    )(num_parts, den_col)
    return o
